# baseline (speedup 1.0000x reference)
import functools

import jax
import jax.numpy as jnp
from jax import lax
from jax.experimental import pallas as pl
from jax.experimental.pallas import tpu as pltpu
from jax.experimental.pallas import tpu_sc as plsc

NUM_EMBEDDINGS = 1000000
EMBEDDING_DIM = 32
HASH_BUCKETS = 100000
BATCH = 4096
SEQ = 200

NC = 2
NS = 16
L = 16
NW = NC * NS

HALF = 8
BATCH_H = BATCH // HALF
ROWS_W = BATCH_H // NW
RB = 4
CHUNK = RB * SEQ
NCHUNK = ROWS_W // RB

MULT_MOD = 2654435761 % HASH_BUCKETS
MULT_HI = (128 * MULT_MOD) % HASH_BUCKETS


def _i32(v):
    return jnp.int32(v)


def _hash16(x):
    a = x % _i32(HASH_BUCKETS)
    hi = lax.shift_right_logical(a, _i32(7))
    lo = lax.bitwise_and(a, _i32(127))
    return (hi * _i32(MULT_HI) % _i32(HASH_BUCKETS)
            + lo * _i32(MULT_MOD)) % _i32(HASH_BUCKETS)


def _sc_body(table_hbm, ids_hbm, out_hbm,
             idx_raw0, idx_raw1, idx_hash0, idx_hash1, rows0, rows1,
             isem0, isem1, gsem0, gsem1, osem0, osem1):
    wid = lax.axis_index("s") * _i32(NC) + lax.axis_index("c")
    row_w = wid * _i32(ROWS_W)

    idx_raw = (idx_raw0, idx_raw1)
    idx_hash = (idx_hash0, idx_hash1)
    rows = (rows0, rows1)
    isem = (isem0, isem1)
    gsem = (gsem0, gsem1)
    osem = (osem0, osem1)

    def ids_slice(c):
        base = (row_w + c * _i32(RB)) * _i32(SEQ)
        return ids_hbm.at[pl.ds(base, CHUNK)]

    def issue_in(c, s):
        pltpu.async_copy(ids_slice(c), idx_raw[s], isem[s])

    def wait_in(c, s):
        pltpu.make_async_copy(ids_slice(c), idx_raw[s], isem[s]).wait()

    def do_hash(s):
        ir, ih = idx_raw[s], idx_hash[s]

        def hash_body(j, _):
            ih[pl.ds(j * _i32(L), L)] = _hash16(ir[pl.ds(j * _i32(L), L)])
            return 0

        lax.fori_loop(_i32(0), _i32(CHUNK // L), hash_body, 0)

    def issue_gather(s):
        pltpu.async_copy(table_hbm.at[idx_hash[s]], rows[s], gsem[s])

    def wait_gather(s):
        pltpu.make_async_copy(table_hbm.at[idx_hash[s]], rows[s],
                              gsem[s]).wait()

    def issue_out(c, s):
        row0 = row_w + c * _i32(RB)
        for r in range(RB):
            pltpu.async_copy(rows[s].at[pl.ds(r * SEQ, SEQ)],
                             out_hbm.at[row0 + _i32(r)], osem[s])

    def wait_out(c, s):
        row0 = row_w + c * _i32(RB)
        for r in range(RB):
            pltpu.make_async_copy(rows[s].at[pl.ds(r * SEQ, SEQ)],
                                  out_hbm.at[row0 + _i32(r)],
                                  osem[s]).wait()

    def steady(c, s):
        wait_in(c, s)
        do_hash(s)
        wait_out(c - _i32(2), s)
        issue_gather(s)
        issue_in(c + _i32(1), 1 - s)
        wait_gather(1 - s)
        issue_out(c - _i32(1), 1 - s)

    issue_in(_i32(0), 0)
    wait_in(_i32(0), 0)
    do_hash(0)
    issue_gather(0)
    issue_in(_i32(1), 1)

    wait_in(_i32(1), 1)
    do_hash(1)
    issue_gather(1)
    issue_in(_i32(2), 0)
    wait_gather(0)
    issue_out(_i32(0), 0)

    def pair_body(c2, _):
        c = c2 * _i32(2)
        steady(c, 0)
        steady(c + _i32(1), 1)
        return 0

    lax.fori_loop(_i32(1), _i32(NCHUNK // 2 - 1), pair_body, 0)

    steady(_i32(NCHUNK - 2), 0)

    cl = _i32(NCHUNK - 1)
    wait_in(cl, 1)
    do_hash(1)
    wait_out(cl - _i32(2), 1)
    issue_gather(1)
    wait_gather(0)
    issue_out(cl - _i32(1), 0)

    wait_gather(1)
    issue_out(cl, 1)
    wait_out(cl - _i32(1), 0)
    wait_out(cl, 1)


@jax.jit
def _hashed_lookup_half(ids_half, table):
    mesh = plsc.VectorSubcoreMesh(core_axis_name="c", subcore_axis_name="s")
    k = functools.partial(
        pl.kernel,
        out_type=jax.ShapeDtypeStruct((BATCH_H, SEQ, EMBEDDING_DIM),
                                      jnp.float32),
        mesh=mesh,
        scratch_types=[
            pltpu.VMEM((CHUNK,), jnp.int32),
            pltpu.VMEM((CHUNK,), jnp.int32),
            pltpu.VMEM((CHUNK,), jnp.int32),
            pltpu.VMEM((CHUNK,), jnp.int32),
            pltpu.VMEM((CHUNK, EMBEDDING_DIM), jnp.float32),
            pltpu.VMEM((CHUNK, EMBEDDING_DIM), jnp.float32),
            pltpu.SemaphoreType.DMA,
            pltpu.SemaphoreType.DMA,
            pltpu.SemaphoreType.DMA,
            pltpu.SemaphoreType.DMA,
            pltpu.SemaphoreType.DMA,
            pltpu.SemaphoreType.DMA,
        ],
        compiler_params=pltpu.CompilerParams(use_tc_tiling_on_sc=False),
    )(_sc_body)
    return k(table, ids_half)


def kernel(input_ids, embedding_table, hash_weights):
    ids32 = input_ids.reshape(-1).astype(jnp.int32)
    n_half = BATCH_H * SEQ
    halves = [
        _hashed_lookup_half(ids32[h * n_half:(h + 1) * n_half],
                            embedding_table)
        for h in range(HALF)
    ]
    return jnp.concatenate(halves, axis=0)


# speedup vs baseline: 1.0305x; 1.0305x over previous
import functools

import jax
import jax.numpy as jnp
from jax import lax
from jax.experimental import pallas as pl
from jax.experimental.pallas import tpu as pltpu
from jax.experimental.pallas import tpu_sc as plsc

NUM_EMBEDDINGS = 1000000
EMBEDDING_DIM = 32
HASH_BUCKETS = 100000
BATCH = 4096
SEQ = 200

NC = 2
NS = 16
L = 16
NW = NC * NS

HALF = 4
BATCH_H = BATCH // HALF
ROWS_W = BATCH_H // NW
RB = 4
CHUNK = RB * SEQ
NCHUNK = ROWS_W // RB

MULT_MOD = 2654435761 % HASH_BUCKETS
MULT_HI = (128 * MULT_MOD) % HASH_BUCKETS


def _i32(v):
    return jnp.int32(v)


def _hash16(x):
    a = x % _i32(HASH_BUCKETS)
    hi = lax.shift_right_logical(a, _i32(7))
    lo = lax.bitwise_and(a, _i32(127))
    return (hi * _i32(MULT_HI) % _i32(HASH_BUCKETS)
            + lo * _i32(MULT_MOD)) % _i32(HASH_BUCKETS)


def _sc_body(table_hbm, ids_hbm, out_hbm,
             idx_raw0, idx_raw1, idx_hash0, idx_hash1, rows0, rows1,
             isem0, isem1, gsem0, gsem1, osem0, osem1):
    wid = lax.axis_index("s") * _i32(NC) + lax.axis_index("c")
    row_w = wid * _i32(ROWS_W)

    idx_raw = (idx_raw0, idx_raw1)
    idx_hash = (idx_hash0, idx_hash1)
    rows = (rows0, rows1)
    isem = (isem0, isem1)
    gsem = (gsem0, gsem1)
    osem = (osem0, osem1)

    def ids_slice(c):
        base = (row_w + c * _i32(RB)) * _i32(SEQ)
        return ids_hbm.at[pl.ds(base, CHUNK)]

    def issue_in(c, s):
        pltpu.async_copy(ids_slice(c), idx_raw[s], isem[s])

    def wait_in(c, s):
        pltpu.make_async_copy(ids_slice(c), idx_raw[s], isem[s]).wait()

    def do_hash(s):
        ir, ih = idx_raw[s], idx_hash[s]

        def hash_body(j, _):
            ih[pl.ds(j * _i32(L), L)] = _hash16(ir[pl.ds(j * _i32(L), L)])
            return 0

        lax.fori_loop(_i32(0), _i32(CHUNK // L), hash_body, 0)

    def issue_gather(s):
        pltpu.async_copy(table_hbm.at[idx_hash[s]], rows[s], gsem[s])

    def wait_gather(s):
        pltpu.make_async_copy(table_hbm.at[idx_hash[s]], rows[s],
                              gsem[s]).wait()

    def issue_out(c, s):
        row0 = row_w + c * _i32(RB)
        for r in range(RB):
            pltpu.async_copy(rows[s].at[pl.ds(r * SEQ, SEQ)],
                             out_hbm.at[row0 + _i32(r)], osem[s])

    def wait_out(c, s):
        row0 = row_w + c * _i32(RB)
        for r in range(RB):
            pltpu.make_async_copy(rows[s].at[pl.ds(r * SEQ, SEQ)],
                                  out_hbm.at[row0 + _i32(r)],
                                  osem[s]).wait()

    def steady(c, s):
        wait_in(c, s)
        do_hash(s)
        wait_out(c - _i32(2), s)
        issue_gather(s)
        issue_in(c + _i32(1), 1 - s)
        wait_gather(1 - s)
        issue_out(c - _i32(1), 1 - s)

    issue_in(_i32(0), 0)
    wait_in(_i32(0), 0)
    do_hash(0)
    issue_gather(0)
    issue_in(_i32(1), 1)

    wait_in(_i32(1), 1)
    do_hash(1)
    issue_gather(1)
    issue_in(_i32(2), 0)
    wait_gather(0)
    issue_out(_i32(0), 0)

    def pair_body(c2, _):
        c = c2 * _i32(2)
        steady(c, 0)
        steady(c + _i32(1), 1)
        return 0

    lax.fori_loop(_i32(1), _i32(NCHUNK // 2 - 1), pair_body, 0)

    steady(_i32(NCHUNK - 2), 0)

    cl = _i32(NCHUNK - 1)
    wait_in(cl, 1)
    do_hash(1)
    wait_out(cl - _i32(2), 1)
    issue_gather(1)
    wait_gather(0)
    issue_out(cl - _i32(1), 0)

    wait_gather(1)
    issue_out(cl, 1)
    wait_out(cl - _i32(1), 0)
    wait_out(cl, 1)


@jax.jit
def _hashed_lookup_half(ids_half, table):
    mesh = plsc.VectorSubcoreMesh(core_axis_name="c", subcore_axis_name="s")
    k = functools.partial(
        pl.kernel,
        out_type=jax.ShapeDtypeStruct((BATCH_H, SEQ, EMBEDDING_DIM),
                                      jnp.float32),
        mesh=mesh,
        scratch_types=[
            pltpu.VMEM((CHUNK,), jnp.int32),
            pltpu.VMEM((CHUNK,), jnp.int32),
            pltpu.VMEM((CHUNK,), jnp.int32),
            pltpu.VMEM((CHUNK,), jnp.int32),
            pltpu.VMEM((CHUNK, EMBEDDING_DIM), jnp.float32),
            pltpu.VMEM((CHUNK, EMBEDDING_DIM), jnp.float32),
            pltpu.SemaphoreType.DMA,
            pltpu.SemaphoreType.DMA,
            pltpu.SemaphoreType.DMA,
            pltpu.SemaphoreType.DMA,
            pltpu.SemaphoreType.DMA,
            pltpu.SemaphoreType.DMA,
        ],
        compiler_params=pltpu.CompilerParams(use_tc_tiling_on_sc=False),
    )(_sc_body)
    return k(table, ids_half)


def kernel(input_ids, embedding_table, hash_weights):
    ids32 = input_ids.reshape(-1).astype(jnp.int32)
    n_half = BATCH_H * SEQ
    halves = [
        _hashed_lookup_half(ids32[h * n_half:(h + 1) * n_half],
                            embedding_table)
        for h in range(HALF)
    ]
    return jnp.concatenate(halves, axis=0)
